# baseline (speedup 1.0000x reference)
import functools

import jax
import jax.numpy as jnp
from jax import lax
from jax.experimental import pallas as pl
from jax.experimental.pallas import tpu as pltpu
from jax.experimental.pallas import tpu_sc as plsc

_NC = 2
_NS = 16


_CH = 125


def _sc_aggregate(xview, sd, zeros_tile):
    n2, d = xview.shape
    n = n2 // 2
    ch = _CH
    total_chunks = sd.shape[1]
    n_chunks = total_chunks // _NS
    rows_per_tile = n // _NS

    mesh = plsc.VectorSubcoreMesh(
        core_axis_name="c", subcore_axis_name="s", num_cores=_NC,
        num_subcores=_NS)

    nbuf = 5
    n_groups = n_chunks // nbuf

    @functools.partial(
        pl.kernel,
        out_type=jax.ShapeDtypeStruct((n, 2 * d), jnp.float32),
        mesh=mesh,
        scratch_types=[
            pltpu.VMEM((n_chunks, ch), jnp.int32),
            pltpu.VMEM((n_chunks, ch), jnp.int32),
        ] + [pltpu.VMEM((ch, d), jnp.float32) for _ in range(nbuf)] + [
            pltpu.VMEM_SHARED((n, d), jnp.float32),
        ] + [pltpu.SemaphoreType.DMA for _ in range(2 * nbuf)],
        compiler_params=pltpu.CompilerParams(use_tc_tiling_on_sc=False),
    )
    def k(x_hbm, sd_hbm, z_hbm, out_hbm, sidx, didx, *rest):
        rows = rest[:nbuf]
        acc = rest[nbuf]
        gsem = rest[nbuf + 1:nbuf + 1 + nbuf]
        ssem = rest[nbuf + 1 + nbuf:]
        c = lax.axis_index("c")
        s = lax.axis_index("s")

        zcp = pltpu.async_copy(z_hbm, rows[0], gsem[0])
        pltpu.sync_copy(sd_hbm.at[0].at[pl.ds(s * n_chunks, n_chunks)], sidx)
        zcp.wait()
        zs = []
        for r in range(rows_per_tile // ch):
            zs.append(pltpu.async_copy(
                rows[0], acc.at[pl.ds(s * rows_per_tile + r * ch, ch)],
                ssem[r % nbuf]))
        pltpu.sync_copy(sd_hbm.at[1].at[pl.ds(s * n_chunks, n_chunks)], didx)
        for z in zs:
            z.wait()
        plsc.subcore_barrier()
        xs = x_hbm.at[pl.ds(c, n2 - 1)]

        def gather(i, b):
            return pltpu.async_copy(xs.at[sidx.at[i]], rows[b], gsem[b])

        def gather_wait(i, b):
            pltpu.make_async_copy(xs.at[sidx.at[i]], rows[b],
                                  gsem[b]).wait()

        def scat(i, b):
            return pltpu.async_copy(rows[b], acc.at[didx.at[i]], ssem[b],
                                    add=True)

        def scat_wait(i, b):
            pltpu.make_async_copy(rows[b], acc.at[didx.at[i]], ssem[b]).wait()

        for b in range(nbuf):
            gather(b, b)

        def body(j, carry):
            i0 = nbuf * j
            for b in range(nbuf):
                gather_wait(i0 + b, b)
                scat(i0 + b, b)
            for b in range(nbuf):
                scat_wait(i0 + b, b)
                gather(i0 + nbuf + b, b)
            return carry

        lax.fori_loop(0, n_groups - 1, body, 0)
        ilast = nbuf * (n_groups - 1)
        for b in range(nbuf):
            gather_wait(ilast + b, b)
            scat(ilast + b, b)
        for b in range(nbuf):
            scat_wait(ilast + b, b)
        plsc.subcore_barrier()

        r0 = s * rows_per_tile
        pltpu.sync_copy(acc.at[pl.ds(r0, rows_per_tile)],
                        out_hbm.at[pl.ds(r0, rows_per_tile), pl.ds(c * d, d)])

    return k(xview, sd, zeros_tile)


def _layer_core(x_ref, p_ref, scale_ref, w1_ref, b1_ref, w2_ref, b2_ref,
                g_ref, be_ref):
    n = x_ref.shape[0]
    h0 = scale_ref[0, 0] * x_ref[...] + p_ref[...]
    a = jnp.maximum(
        jnp.dot(h0, w1_ref[...], preferred_element_type=jnp.float32)
        + b1_ref[...], 0.0)
    t = jnp.maximum(
        jnp.dot(a, w2_ref[...], preferred_element_type=jnp.float32)
        + b2_ref[...], 0.0)
    mean = jnp.sum(t, axis=0, keepdims=True) / n
    var = jnp.sum(t * t, axis=0, keepdims=True) / n - mean * mean
    inv = lax.rsqrt(var + 1e-5)
    h = g_ref[...] * (t - mean) * inv + be_ref[...]
    return jnp.maximum(h, 0.0)


def _layer_body(x_ref, p_ref, scale_ref, w1_ref, b1_ref, w2_ref, b2_ref,
                g_ref, be_ref, h_ref):
    h_ref[...] = _layer_core(x_ref, p_ref, scale_ref, w1_ref, b1_ref, w2_ref,
                             b2_ref, g_ref, be_ref)


def _layer_out_body(x_ref, p_ref, scale_ref, w1_ref, b1_ref, w2_ref, b2_ref,
                    g_ref, be_ref, wo_ref, bo_ref, o_ref):
    h = _layer_core(x_ref, p_ref, scale_ref, w1_ref, b1_ref, w2_ref, b2_ref,
                    g_ref, be_ref)
    o_ref[...] = (
        jnp.dot(h, wo_ref[...], preferred_element_type=jnp.float32)
        + bo_ref[...])


def _smem_spec():
    return pl.BlockSpec(memory_space=pltpu.SMEM)


def _layer(x, p, scale, w1, b1, w2, b2, g, be):
    n, d = x.shape
    return pl.pallas_call(
        _layer_body,
        in_specs=[pl.BlockSpec((n, d), lambda: (0, 0)),
                  pl.BlockSpec((n, d), lambda: (0, 0)),
                  _smem_spec()] + [pl.BlockSpec(b.shape, lambda: (0, 0))
                                   for b in (w1, b1, w2, b2, g, be)],
        out_specs=pl.BlockSpec((n, d), lambda: (0, 0)),
        out_shape=jax.ShapeDtypeStruct((n, d), jnp.float32),
    )(x, p, scale, w1, b1, w2, b2, g, be)


def _layer_out(x, p, scale, w1, b1, w2, b2, g, be, wo, bo):
    n, d = x.shape
    dout = wo.shape[1]
    return pl.pallas_call(
        _layer_out_body,
        in_specs=[pl.BlockSpec((n, d), lambda: (0, 0)),
                  pl.BlockSpec((n, d), lambda: (0, 0)),
                  _smem_spec()] + [pl.BlockSpec(b.shape, lambda: (0, 0))
                                   for b in (w1, b1, w2, b2, g, be, wo, bo)],
        out_specs=pl.BlockSpec((n, dout), lambda: (0, 0)),
        out_shape=jax.ShapeDtypeStruct((n, dout), jnp.float32),
    )(x, p, scale, w1, b1, w2, b2, g, be, wo, bo)


@jax.jit
def kernel(x, edge_index, eps1, W11, b11, W12, b12, g1, be1, eps2, W21, b21,
           W22, b22, g2, be2, Wo, bo):
    n, d = x.shape
    zeros_tile = jnp.zeros((_CH, d // 2), jnp.float32)
    sd = (edge_index * jnp.array([[2], [1]], jnp.int32)).reshape(2, -1, _CH)

    r2 = lambda v: v.reshape(1, -1)
    scale1 = (1.0 + eps1).reshape(1, 1)
    scale2 = (1.0 + eps2).reshape(1, 1)

    def agg(v):
        return _sc_aggregate(v.reshape(2 * n, d // 2), sd, zeros_tile)

    p1 = agg(x)
    h1 = _layer(x, p1, scale1, W11, r2(b11), W12, r2(b12), r2(g1), r2(be1))
    p2 = agg(h1)
    out = _layer_out(h1, p2, scale2, W21, r2(b21), W22, r2(b22), r2(g2),
                     r2(be2), Wo, r2(bo))
    return out
